# baseline (speedup 1.0000x reference)
import jax
import jax.numpy as jnp
from jax import lax
from jax.experimental import pallas as pl
from jax.experimental.pallas import tpu as pltpu
from jax.experimental.pallas import tpu_sc as plsc

N_NODES = 10000
N_EDGES = 320000
D = 128

NC = 2
NS = 16
E_PER_TILE = N_EDGES // (NC * NS)
CHUNK = 128
EPT_PAD = 10240
N_CHUNKS = EPT_PAD // CHUNK
N_PAD = 10240
OWN1D = N_PAD // NS

_MESH = plsc.VectorSubcoreMesh(core_axis_name="c", subcore_axis_name="s")


def _zero_1d(buf, n_vecs):
    zeros = jnp.zeros((16,), jnp.float32)

    def body(i, _):
        buf[pl.ds(i * 16, 16)] = zeros
        return 0

    lax.fori_loop(0, n_vecs, body, 0)


def _zero_rows(buf, n_rows):
    zeros = jnp.zeros((16,), jnp.float32)

    def body(e, _):
        for j in range(8):
            buf[e, pl.ds(j * 16, 16)] = zeros
        return 0

    lax.fori_loop(0, n_rows, body, 0)


def _acc1d_init(s, flat, acc):
    _zero_1d(flat, OWN1D // 16)
    pltpu.sync_copy(flat.at[pl.ds(0, OWN1D)],
                    acc.at[pl.ds(s * OWN1D, OWN1D)])


def _acc1d_dump(c, s, acc, out_hbm):
    pltpu.sync_copy(acc.at[pl.ds(s * OWN1D, OWN1D)],
                    out_hbm.at[c, pl.ds(s * OWN1D, OWN1D)])


def _deg_body(col3_hbm, ew3_hbm, degp_hbm, colv, eww, flat, acc):
    c = lax.axis_index("c")
    s = lax.axis_index("s")
    wid = c * NS + s

    pltpu.sync_copy(col3_hbm.at[wid], colv)
    pltpu.sync_copy(ew3_hbm.at[wid], eww)
    _acc1d_init(s, flat, acc)
    plsc.subcore_barrier()

    def body(i, _):
        pltpu.sync_copy(eww.at[i], acc.at[colv.at[i]], add=True)
        return 0

    lax.fori_loop(0, N_CHUNKS, body, 0)
    plsc.subcore_barrier()
    _acc1d_dump(c, s, acc, degp_hbm)


_deg_pass = pl.kernel(
    _deg_body,
    out_type=jax.ShapeDtypeStruct((NC, N_PAD), jnp.float32),
    mesh=_MESH,
    scratch_types=dict(
        colv=pltpu.VMEM((N_CHUNKS, CHUNK), jnp.int32),
        eww=pltpu.VMEM((N_CHUNKS, CHUNK), jnp.float32),
        flat=pltpu.VMEM((OWN1D,), jnp.float32),
        acc=pltpu.VMEM_SHARED((N_PAD,), jnp.float32),
    ),
)


SUP = 16
N_SUP = N_CHUNKS // SUP


def _agg_body(xw2_hbm, row3_hbm, col3_hbm, ew3_hbm, sp_hbm, rowv, colv, eww,
              buf0, buf1, sem0, sem1, ssem0, ssem1, acc):
    c = lax.axis_index("c")
    s = lax.axis_index("s")
    wid = c * NS + s

    _zero_rows(buf0, CHUNK)
    r0 = s * (N_PAD // NS)
    for k in range(OWN1D // CHUNK):
        pltpu.sync_copy(buf0, acc.at[pl.ds(r0 + k * CHUNK, CHUNK)])
    plsc.subcore_barrier()

    def scale(buf, i):
        @plsc.parallel_loop(0, CHUNK // 16, 1, unroll=2)
        def g_body(g):
            w16 = eww[i, pl.ds(g * 16, 16)]
            for kk in range(16):
                w = jnp.full((16,), w16[kk], jnp.float32)
                e = g * 16 + kk
                for j in range(8):
                    sl = pl.ds(j * 16, 16)
                    buf[e, sl] = buf[e, sl] * w

    def super_body(u, _):
        u0 = u * SUP
        pltpu.sync_copy(row3_hbm.at[wid, pl.ds(u0, SUP)], rowv)
        pltpu.sync_copy(col3_hbm.at[wid, pl.ds(u0, SUP)], colv)
        pltpu.sync_copy(ew3_hbm.at[wid, pl.ds(u0, SUP)], eww)

        pltpu.async_copy(xw2_hbm.at[rowv.at[0]], buf0, sem0)
        pltpu.async_copy(xw2_hbm.at[rowv.at[1]], buf1, sem1)

        def body(k, _):
            i0 = 2 * k
            i1 = 2 * k + 1
            pltpu.make_async_copy(xw2_hbm.at[rowv.at[i0]], buf0, sem0).wait()
            scale(buf0, i0)
            pltpu.async_copy(buf0, acc.at[colv.at[i0]], ssem0, add=True)

            pltpu.make_async_copy(xw2_hbm.at[rowv.at[i1]], buf1, sem1).wait()
            scale(buf1, i1)
            pltpu.async_copy(buf1, acc.at[colv.at[i1]], ssem1, add=True)

            pltpu.make_async_copy(buf0, acc.at[colv.at[i0]], ssem0).wait()

            @pl.when(k < SUP // 2 - 1)
            def _():
                pltpu.async_copy(xw2_hbm.at[rowv.at[i0 + 2]], buf0, sem0)

            pltpu.make_async_copy(buf1, acc.at[colv.at[i1]], ssem1).wait()

            @pl.when(k < SUP // 2 - 1)
            def _():
                pltpu.async_copy(xw2_hbm.at[rowv.at[i1 + 2]], buf1, sem1)

            return 0

        lax.fori_loop(0, SUP // 2, body, 0)
        return 0

    lax.fori_loop(0, N_SUP, super_body, 0)
    plsc.subcore_barrier()
    for k in range(OWN1D // CHUNK):
        pltpu.sync_copy(acc.at[pl.ds(r0 + k * CHUNK, CHUNK)],
                        sp_hbm.at[c, pl.ds(r0 + k * CHUNK, CHUNK)])


_agg_pass = pl.kernel(
    _agg_body,
    out_type=jax.ShapeDtypeStruct((NC, N_PAD, D), jnp.float32),
    mesh=_MESH,
    scratch_types=dict(
        rowv=pltpu.VMEM((SUP, CHUNK), jnp.int32),
        colv=pltpu.VMEM((SUP, CHUNK), jnp.int32),
        eww=pltpu.VMEM((SUP, CHUNK), jnp.float32),
        buf0=pltpu.VMEM((CHUNK, D), jnp.float32),
        buf1=pltpu.VMEM((CHUNK, D), jnp.float32),
        sem0=pltpu.SemaphoreType.DMA,
        sem1=pltpu.SemaphoreType.DMA,
        ssem0=pltpu.SemaphoreType.DMA,
        ssem1=pltpu.SemaphoreType.DMA,
        acc=pltpu.VMEM_SHARED((N_PAD, D), jnp.float32),
    ),
)


def _sagg_body(z2_hbm, row3_hbm, col3_hbm, ew3_hbm, tp_hbm, rowv, colv, eww,
               val0, val1, sem0, sem1, ssem0, ssem1, flat, acc):
    c = lax.axis_index("c")
    s = lax.axis_index("s")
    wid = c * NS + s

    pltpu.sync_copy(row3_hbm.at[wid], rowv)
    pltpu.sync_copy(col3_hbm.at[wid], colv)
    pltpu.sync_copy(ew3_hbm.at[wid], eww)
    _acc1d_init(s, flat, acc)
    plsc.subcore_barrier()

    def scale(buf, i):
        @plsc.parallel_loop(0, CHUNK // 16, 1, unroll=4)
        def g_body(g):
            sl = pl.ds(g * 16, 16)
            buf[sl] = buf[sl] * eww[i, sl]

    pltpu.async_copy(z2_hbm.at[rowv.at[0]], val0, sem0)
    pltpu.async_copy(z2_hbm.at[rowv.at[1]], val1, sem1)

    def body(k, _):
        i0 = 2 * k
        i1 = 2 * k + 1
        pltpu.make_async_copy(z2_hbm.at[rowv.at[i0]], val0, sem0).wait()
        scale(val0, i0)
        pltpu.async_copy(val0, acc.at[colv.at[i0]], ssem0, add=True)

        pltpu.make_async_copy(z2_hbm.at[rowv.at[i1]], val1, sem1).wait()
        scale(val1, i1)
        pltpu.async_copy(val1, acc.at[colv.at[i1]], ssem1, add=True)

        pltpu.make_async_copy(val0, acc.at[colv.at[i0]], ssem0).wait()

        @pl.when(k < N_CHUNKS // 2 - 1)
        def _():
            pltpu.async_copy(z2_hbm.at[rowv.at[i0 + 2]], val0, sem0)

        pltpu.make_async_copy(val1, acc.at[colv.at[i1]], ssem1).wait()

        @pl.when(k < N_CHUNKS // 2 - 1)
        def _():
            pltpu.async_copy(z2_hbm.at[rowv.at[i1 + 2]], val1, sem1)

        return 0

    lax.fori_loop(0, N_CHUNKS // 2, body, 0)
    plsc.subcore_barrier()
    _acc1d_dump(c, s, acc, tp_hbm)


_sagg_pass = pl.kernel(
    _sagg_body,
    out_type=jax.ShapeDtypeStruct((NC, N_PAD), jnp.float32),
    mesh=_MESH,
    scratch_types=dict(
        rowv=pltpu.VMEM((N_CHUNKS, CHUNK), jnp.int32),
        colv=pltpu.VMEM((N_CHUNKS, CHUNK), jnp.int32),
        eww=pltpu.VMEM((N_CHUNKS, CHUNK), jnp.float32),
        val0=pltpu.VMEM((CHUNK,), jnp.float32),
        val1=pltpu.VMEM((CHUNK,), jnp.float32),
        sem0=pltpu.SemaphoreType.DMA,
        sem1=pltpu.SemaphoreType.DMA,
        ssem0=pltpu.SemaphoreType.DMA,
        ssem1=pltpu.SemaphoreType.DMA,
        flat=pltpu.VMEM((OWN1D,), jnp.float32),
        acc=pltpu.VMEM_SHARED((N_PAD,), jnp.float32),
    ),
)


_BM = 2000


def _mm_body(x_ref, w_ref, o_ref):
    o_ref[...] = jnp.dot(x_ref[...], w_ref[...],
                         preferred_element_type=jnp.float32)


def _tc_matmul(x, w):
    return pl.pallas_call(
        _mm_body,
        grid=(N_NODES // _BM,),
        in_specs=[
            pl.BlockSpec((_BM, D), lambda i: (i, 0)),
            pl.BlockSpec((D, D), lambda i: (0, 0)),
        ],
        out_specs=pl.BlockSpec((_BM, D), lambda i: (i, 0)),
        out_shape=jax.ShapeDtypeStruct((N_NODES, D), jnp.float32),
    )(x, w)


def _scale_body(degp_ref, xw_ref, dis_ref, xw2_ref):
    deg = degp_ref[0, :, :] + degp_ref[1, :, :] + 1.0
    dis = lax.rsqrt(deg)
    dis_ref[...] = dis
    xw2_ref[...] = xw_ref[...] * dis


def _tc_scale(degp, xw):
    return pl.pallas_call(
        _scale_body,
        grid=(N_NODES // _BM,),
        in_specs=[
            pl.BlockSpec((NC, _BM, 1), lambda i: (0, i, 0)),
            pl.BlockSpec((_BM, D), lambda i: (i, 0)),
        ],
        out_specs=[
            pl.BlockSpec((_BM, 1), lambda i: (i, 0)),
            pl.BlockSpec((_BM, D), lambda i: (i, 0)),
        ],
        out_shape=[
            jax.ShapeDtypeStruct((N_NODES, 1), jnp.float32),
            jax.ShapeDtypeStruct((N_NODES, D), jnp.float32),
        ],
    )(degp.reshape(NC, N_NODES, 1), xw)


def _layer2_body(sp_ref, xw2_ref, dis_ref, b1_ref, w2_ref, z2_ref):
    dis = dis_ref[...]
    h = sp_ref[0, :, :] + sp_ref[1, :, :] + xw2_ref[...]
    h = jnp.maximum(h * dis + b1_ref[...], 0.0)
    z = jnp.dot(h, w2_ref[...], preferred_element_type=jnp.float32)
    z2_ref[...] = z * dis


def _tc_layer2(sp, xw2, dis, b1, w2):
    return pl.pallas_call(
        _layer2_body,
        grid=(N_NODES // _BM,),
        in_specs=[
            pl.BlockSpec((NC, _BM, D), lambda i: (0, i, 0)),
            pl.BlockSpec((_BM, D), lambda i: (i, 0)),
            pl.BlockSpec((_BM, 1), lambda i: (i, 0)),
            pl.BlockSpec((1, D), lambda i: (0, 0)),
            pl.BlockSpec((D, 1), lambda i: (0, 0)),
        ],
        out_specs=pl.BlockSpec((_BM, 1), lambda i: (i, 0)),
        out_shape=jax.ShapeDtypeStruct((N_NODES, 1), jnp.float32),
    )(sp, xw2, dis, b1.reshape(1, D), w2)


def _final_body(tp_ref, z2_ref, dis_ref, b2_ref, o_ref):
    t = tp_ref[0, :, :] + tp_ref[1, :, :] + z2_ref[...]
    o_ref[...] = t * dis_ref[...] + b2_ref[...]


def _tc_final(tp, z2, dis, b2):
    return pl.pallas_call(
        _final_body,
        grid=(N_NODES // _BM,),
        in_specs=[
            pl.BlockSpec((NC, _BM, 1), lambda i: (0, i, 0)),
            pl.BlockSpec((_BM, 1), lambda i: (i, 0)),
            pl.BlockSpec((_BM, 1), lambda i: (i, 0)),
            pl.BlockSpec((1, 1), lambda i: (0, 0)),
        ],
        out_specs=pl.BlockSpec((_BM, 1), lambda i: (i, 0)),
        out_shape=jax.ShapeDtypeStruct((N_NODES, 1), jnp.float32),
    )(tp.reshape(NC, N_NODES, 1), z2, dis, b2.reshape(1, 1))


def _pad_edges(v, fill):
    v = v.reshape(NC * NS, E_PER_TILE)
    pad = jnp.full((NC * NS, EPT_PAD - E_PER_TILE), fill, v.dtype)
    return jnp.concatenate([v, pad], axis=1).reshape(NC * NS, N_CHUNKS, CHUNK)


@jax.jit
def kernel(x, edge_index, edge_attr, W1, b1, W2, b2):
    row = edge_index[0].astype(jnp.int32)
    col = edge_index[1].astype(jnp.int32)
    ew = edge_attr[:, 0]
    row3 = _pad_edges(row, 0)
    col3 = _pad_edges(col, N_PAD - 1)
    ew3 = _pad_edges(ew, 0.0)

    degp = _deg_pass(col3, ew3)[:, :N_NODES]
    xw = _tc_matmul(x, W1)
    dis, xw2 = _tc_scale(degp, xw)
    sp = _agg_pass(xw2, row3, col3, ew3)[:, :N_NODES]
    z2 = _tc_layer2(sp, xw2, dis, b1, W2)
    tp = _sagg_pass(z2[:, 0], row3, col3, ew3)[:, :N_NODES]
    out = _tc_final(tp, z2, dis, b2)
    return out[:, 0]


# speedup vs baseline: 1.0212x; 1.0212x over previous
import jax
import jax.numpy as jnp
from jax import lax
from jax.experimental import pallas as pl
from jax.experimental.pallas import tpu as pltpu
from jax.experimental.pallas import tpu_sc as plsc

N_NODES = 10000
N_EDGES = 320000
D = 128

NC = 2
NS = 16
E_PER_TILE = N_EDGES // (NC * NS)
CHUNK = 128
EPT_PAD = 10240
N_CHUNKS = EPT_PAD // CHUNK
N_PAD = 10240
OWN1D = N_PAD // NS

_MESH = plsc.VectorSubcoreMesh(core_axis_name="c", subcore_axis_name="s")


def _zero_1d(buf, n_vecs):
    zeros = jnp.zeros((16,), jnp.float32)

    def body(i, _):
        buf[pl.ds(i * 16, 16)] = zeros
        return 0

    lax.fori_loop(0, n_vecs, body, 0)


def _zero_rows(buf, n_rows):
    zeros = jnp.zeros((16,), jnp.float32)

    def body(e, _):
        for j in range(8):
            buf[e, pl.ds(j * 16, 16)] = zeros
        return 0

    lax.fori_loop(0, n_rows, body, 0)


def _acc1d_init(s, flat, acc):
    _zero_1d(flat, OWN1D // 16)
    pltpu.sync_copy(flat.at[pl.ds(0, OWN1D)],
                    acc.at[pl.ds(s * OWN1D, OWN1D)])


def _acc1d_dump(c, s, acc, out_hbm):
    pltpu.sync_copy(acc.at[pl.ds(s * OWN1D, OWN1D)],
                    out_hbm.at[c, pl.ds(s * OWN1D, OWN1D)])


def _deg_body(col3_hbm, ew3_hbm, degp_hbm, colv, eww, flat, acc):
    c = lax.axis_index("c")
    s = lax.axis_index("s")
    wid = c * NS + s

    pltpu.sync_copy(col3_hbm.at[wid], colv)
    pltpu.sync_copy(ew3_hbm.at[wid], eww)
    _acc1d_init(s, flat, acc)
    plsc.subcore_barrier()

    def body(i, _):
        pltpu.sync_copy(eww.at[i], acc.at[colv.at[i]], add=True)
        return 0

    lax.fori_loop(0, N_CHUNKS, body, 0)
    plsc.subcore_barrier()
    _acc1d_dump(c, s, acc, degp_hbm)


_deg_pass = pl.kernel(
    _deg_body,
    out_type=jax.ShapeDtypeStruct((NC, N_PAD), jnp.float32),
    mesh=_MESH,
    scratch_types=dict(
        colv=pltpu.VMEM((N_CHUNKS, CHUNK), jnp.int32),
        eww=pltpu.VMEM((N_CHUNKS, CHUNK), jnp.float32),
        flat=pltpu.VMEM((OWN1D,), jnp.float32),
        acc=pltpu.VMEM_SHARED((N_PAD,), jnp.float32),
    ),
)


SUP = 16
N_SUP = N_CHUNKS // SUP


def _agg_body(xw2_hbm, row3_hbm, col3_hbm, ew3_hbm, sp_hbm, rowv, colv, eww,
              buf0, buf1, sem0, sem1, acc):
    c = lax.axis_index("c")
    s = lax.axis_index("s")
    wid = c * NS + s

    _zero_rows(buf0, CHUNK)
    r0 = s * (N_PAD // NS)
    for k in range(OWN1D // CHUNK):
        pltpu.sync_copy(buf0, acc.at[pl.ds(r0 + k * CHUNK, CHUNK)])
    plsc.subcore_barrier()

    def scale(buf, i):
        def g_body(g, _):
            w16 = eww[i, pl.ds(g * 16, 16)]
            for kk in range(16):
                w = jnp.full((16,), w16[kk], jnp.float32)
                e = g * 16 + kk
                for j in range(8):
                    sl = pl.ds(j * 16, 16)
                    buf[e, sl] = buf[e, sl] * w
            return 0

        lax.fori_loop(0, CHUNK // 16, g_body, 0)

    def super_body(u, _):
        u0 = u * SUP
        pltpu.sync_copy(row3_hbm.at[wid, pl.ds(u0, SUP)], rowv)
        pltpu.sync_copy(col3_hbm.at[wid, pl.ds(u0, SUP)], colv)
        pltpu.sync_copy(ew3_hbm.at[wid, pl.ds(u0, SUP)], eww)

        pltpu.async_copy(xw2_hbm.at[rowv.at[0]], buf0, sem0)

        def body(k, _):
            i0 = 2 * k
            i1 = 2 * k + 1
            pltpu.async_copy(xw2_hbm.at[rowv.at[i1]], buf1, sem1)
            pltpu.make_async_copy(xw2_hbm.at[rowv.at[i0]], buf0, sem0).wait()
            scale(buf0, i0)
            pltpu.sync_copy(buf0, acc.at[colv.at[i0]], add=True)

            @pl.when(k < SUP // 2 - 1)
            def _():
                pltpu.async_copy(xw2_hbm.at[rowv.at[i0 + 2]], buf0, sem0)

            pltpu.make_async_copy(xw2_hbm.at[rowv.at[i1]], buf1, sem1).wait()
            scale(buf1, i1)
            pltpu.sync_copy(buf1, acc.at[colv.at[i1]], add=True)
            return 0

        lax.fori_loop(0, SUP // 2, body, 0)
        return 0

    lax.fori_loop(0, N_SUP, super_body, 0)
    plsc.subcore_barrier()
    for k in range(OWN1D // CHUNK):
        pltpu.sync_copy(acc.at[pl.ds(r0 + k * CHUNK, CHUNK)],
                        sp_hbm.at[c, pl.ds(r0 + k * CHUNK, CHUNK)])


_agg_pass = pl.kernel(
    _agg_body,
    out_type=jax.ShapeDtypeStruct((NC, N_PAD, D), jnp.float32),
    mesh=_MESH,
    scratch_types=dict(
        rowv=pltpu.VMEM((SUP, CHUNK), jnp.int32),
        colv=pltpu.VMEM((SUP, CHUNK), jnp.int32),
        eww=pltpu.VMEM((SUP, CHUNK), jnp.float32),
        buf0=pltpu.VMEM((CHUNK, D), jnp.float32),
        buf1=pltpu.VMEM((CHUNK, D), jnp.float32),
        sem0=pltpu.SemaphoreType.DMA,
        sem1=pltpu.SemaphoreType.DMA,
        acc=pltpu.VMEM_SHARED((N_PAD, D), jnp.float32),
    ),
)


def _sagg_body(z2_hbm, row3_hbm, col3_hbm, ew3_hbm, tp_hbm, rowv, colv, eww,
               val0, val1, sem0, sem1, flat, acc):
    c = lax.axis_index("c")
    s = lax.axis_index("s")
    wid = c * NS + s

    pltpu.sync_copy(row3_hbm.at[wid], rowv)
    pltpu.sync_copy(col3_hbm.at[wid], colv)
    pltpu.sync_copy(ew3_hbm.at[wid], eww)
    _acc1d_init(s, flat, acc)
    plsc.subcore_barrier()

    def scale(buf, i):
        def g_body(g, _):
            sl = pl.ds(g * 16, 16)
            buf[sl] = buf[sl] * eww[i, sl]
            return 0

        lax.fori_loop(0, CHUNK // 16, g_body, 0)

    pltpu.async_copy(z2_hbm.at[rowv.at[0]], val0, sem0)

    def body(k, _):
        i0 = 2 * k
        i1 = 2 * k + 1
        pltpu.async_copy(z2_hbm.at[rowv.at[i1]], val1, sem1)
        pltpu.make_async_copy(z2_hbm.at[rowv.at[i0]], val0, sem0).wait()
        scale(val0, i0)
        pltpu.sync_copy(val0, acc.at[colv.at[i0]], add=True)

        @pl.when(k < N_CHUNKS // 2 - 1)
        def _():
            pltpu.async_copy(z2_hbm.at[rowv.at[i0 + 2]], val0, sem0)

        pltpu.make_async_copy(z2_hbm.at[rowv.at[i1]], val1, sem1).wait()
        scale(val1, i1)
        pltpu.sync_copy(val1, acc.at[colv.at[i1]], add=True)
        return 0

    lax.fori_loop(0, N_CHUNKS // 2, body, 0)
    plsc.subcore_barrier()
    _acc1d_dump(c, s, acc, tp_hbm)


_sagg_pass = pl.kernel(
    _sagg_body,
    out_type=jax.ShapeDtypeStruct((NC, N_PAD), jnp.float32),
    mesh=_MESH,
    scratch_types=dict(
        rowv=pltpu.VMEM((N_CHUNKS, CHUNK), jnp.int32),
        colv=pltpu.VMEM((N_CHUNKS, CHUNK), jnp.int32),
        eww=pltpu.VMEM((N_CHUNKS, CHUNK), jnp.float32),
        val0=pltpu.VMEM((CHUNK,), jnp.float32),
        val1=pltpu.VMEM((CHUNK,), jnp.float32),
        sem0=pltpu.SemaphoreType.DMA,
        sem1=pltpu.SemaphoreType.DMA,
        flat=pltpu.VMEM((OWN1D,), jnp.float32),
        acc=pltpu.VMEM_SHARED((N_PAD,), jnp.float32),
    ),
)


_BM = 2000


def _mm_body(x_ref, w_ref, o_ref):
    o_ref[...] = jnp.dot(x_ref[...], w_ref[...],
                         preferred_element_type=jnp.float32)


def _tc_matmul(x, w):
    return pl.pallas_call(
        _mm_body,
        grid=(N_NODES // _BM,),
        in_specs=[
            pl.BlockSpec((_BM, D), lambda i: (i, 0)),
            pl.BlockSpec((D, D), lambda i: (0, 0)),
        ],
        out_specs=pl.BlockSpec((_BM, D), lambda i: (i, 0)),
        out_shape=jax.ShapeDtypeStruct((N_NODES, D), jnp.float32),
    )(x, w)


def _scale_body(degp_ref, xw_ref, dis_ref, xw2_ref):
    deg = degp_ref[0, :, :] + degp_ref[1, :, :] + 1.0
    dis = lax.rsqrt(deg)
    dis_ref[...] = dis
    xw2_ref[...] = xw_ref[...] * dis


def _tc_scale(degp, xw):
    return pl.pallas_call(
        _scale_body,
        grid=(N_NODES // _BM,),
        in_specs=[
            pl.BlockSpec((NC, _BM, 1), lambda i: (0, i, 0)),
            pl.BlockSpec((_BM, D), lambda i: (i, 0)),
        ],
        out_specs=[
            pl.BlockSpec((_BM, 1), lambda i: (i, 0)),
            pl.BlockSpec((_BM, D), lambda i: (i, 0)),
        ],
        out_shape=[
            jax.ShapeDtypeStruct((N_NODES, 1), jnp.float32),
            jax.ShapeDtypeStruct((N_NODES, D), jnp.float32),
        ],
    )(degp.reshape(NC, N_NODES, 1), xw)


def _layer2_body(sp_ref, xw2_ref, dis_ref, b1_ref, w2_ref, z2_ref):
    dis = dis_ref[...]
    h = sp_ref[0, :, :] + sp_ref[1, :, :] + xw2_ref[...]
    h = jnp.maximum(h * dis + b1_ref[...], 0.0)
    z = jnp.dot(h, w2_ref[...], preferred_element_type=jnp.float32)
    z2_ref[...] = z * dis


def _tc_layer2(sp, xw2, dis, b1, w2):
    return pl.pallas_call(
        _layer2_body,
        grid=(N_NODES // _BM,),
        in_specs=[
            pl.BlockSpec((NC, _BM, D), lambda i: (0, i, 0)),
            pl.BlockSpec((_BM, D), lambda i: (i, 0)),
            pl.BlockSpec((_BM, 1), lambda i: (i, 0)),
            pl.BlockSpec((1, D), lambda i: (0, 0)),
            pl.BlockSpec((D, 1), lambda i: (0, 0)),
        ],
        out_specs=pl.BlockSpec((_BM, 1), lambda i: (i, 0)),
        out_shape=jax.ShapeDtypeStruct((N_NODES, 1), jnp.float32),
    )(sp, xw2, dis, b1.reshape(1, D), w2)


def _final_body(tp_ref, z2_ref, dis_ref, b2_ref, o_ref):
    t = tp_ref[0, :, :] + tp_ref[1, :, :] + z2_ref[...]
    o_ref[...] = t * dis_ref[...] + b2_ref[...]


def _tc_final(tp, z2, dis, b2):
    return pl.pallas_call(
        _final_body,
        grid=(N_NODES // _BM,),
        in_specs=[
            pl.BlockSpec((NC, _BM, 1), lambda i: (0, i, 0)),
            pl.BlockSpec((_BM, 1), lambda i: (i, 0)),
            pl.BlockSpec((_BM, 1), lambda i: (i, 0)),
            pl.BlockSpec((1, 1), lambda i: (0, 0)),
        ],
        out_specs=pl.BlockSpec((_BM, 1), lambda i: (i, 0)),
        out_shape=jax.ShapeDtypeStruct((N_NODES, 1), jnp.float32),
    )(tp.reshape(NC, N_NODES, 1), z2, dis, b2.reshape(1, 1))


def _pad_edges(v, fill):
    v = v.reshape(NC * NS, E_PER_TILE)
    pad = jnp.full((NC * NS, EPT_PAD - E_PER_TILE), fill, v.dtype)
    return jnp.concatenate([v, pad], axis=1).reshape(NC * NS, N_CHUNKS, CHUNK)


@jax.jit
def kernel(x, edge_index, edge_attr, W1, b1, W2, b2):
    row = edge_index[0].astype(jnp.int32)
    col = edge_index[1].astype(jnp.int32)
    ew = edge_attr[:, 0]
    row3 = _pad_edges(row, 0)
    col3 = _pad_edges(col, N_PAD - 1)
    ew3 = _pad_edges(ew, 0.0)

    degp = _deg_pass(col3, ew3)[:, :N_NODES]
    xw = _tc_matmul(x, W1)
    dis, xw2 = _tc_scale(degp, xw)
    sp = _agg_pass(xw2, row3, col3, ew3)[:, :N_NODES]
    z2 = _tc_layer2(sp, xw2, dis, b1, W2)
    tp = _sagg_pass(z2[:, 0], row3, col3, ew3)[:, :N_NODES]
    out = _tc_final(tp, z2, dis, b2)
    return out[:, 0]
